# baseline (speedup 1.0000x reference)
import functools
import jax
import jax.numpy as jnp
from jax import lax
from jax.experimental import pallas as pl
from jax.experimental.pallas import tpu as pltpu
from jax.experimental.pallas import tpu_sc as plsc

N = 100000
E = 1600000
D = 32
F_IN = 11
BLK = 1000
NC = 2
NS = 16
NPAD = 100096
NH = 50000
RPT = NH // NS

CH1 = 2000
CH2 = 2000
CHD = 400
CHH = 400

_MESH = plsc.VectorSubcoreMesh(core_axis_name="c", subcore_axis_name="s")
_SC_PARAMS = pltpu.CompilerParams(needs_layout_passes=False, use_tc_tiling_on_sc=False)


def _iota16():
    return lax.iota(jnp.int32, 16)


def _stage_a_body(x_ref, w_ref, asrc_ref, adst_ref, h_ref, al_ref, am_ref):
    i = pl.program_id(0)

    @pl.when(i == 0)
    def _():
        am_ref[...] = jnp.full((1, 2), -jnp.inf, jnp.float32)

    h = jnp.dot(x_ref[...], w_ref[...], preferred_element_type=jnp.float32)
    h_ref[...] = h
    a_s = (h * asrc_ref[...]).sum(-1, keepdims=True)
    a_d = (h * adst_ref[...]).sum(-1, keepdims=True)
    al_ref[:, 0:1] = a_s
    al_ref[:, 1:2] = a_d
    am_ref[0:1, 0:1] = jnp.maximum(am_ref[0:1, 0:1], jnp.max(a_s).reshape(1, 1))
    am_ref[0:1, 1:2] = jnp.maximum(am_ref[0:1, 1:2], jnp.max(a_d).reshape(1, 1))


def _stage_a(x, W_gat, att_src, att_dst):
    return pl.pallas_call(
        _stage_a_body,
        grid=(N // BLK,),
        in_specs=[
            pl.BlockSpec((BLK, F_IN), lambda i: (i, 0)),
            pl.BlockSpec((F_IN, D), lambda i: (0, 0)),
            pl.BlockSpec((1, D), lambda i: (0, 0)),
            pl.BlockSpec((1, D), lambda i: (0, 0)),
        ],
        out_specs=[
            pl.BlockSpec((BLK, D), lambda i: (i, 0)),
            pl.BlockSpec((BLK, 2), lambda i: (i, 0)),
            pl.BlockSpec((1, 2), lambda i: (0, 0)),
        ],
        out_shape=[
            jax.ShapeDtypeStruct((N, D), jnp.float32),
            jax.ShapeDtypeStruct((N, 2), jnp.float32),
            jax.ShapeDtypeStruct((1, 2), jnp.float32),
        ],
    )(x, W_gat, att_src.reshape(1, D), att_dst.reshape(1, D))


def _stage_a2_body(h_ref, al_ref, c_ref, num0_ref, es_ref):
    a = al_ref[:, 0:1] + al_ref[:, 1:2]
    e = jnp.exp(jnp.where(a >= 0, a, 0.2 * a) - c_ref[...])
    es_ref[...] = e
    num0_ref[...] = e * h_ref[...]


def _stage_a2(h, al, c11):
    return pl.pallas_call(
        _stage_a2_body,
        grid=(N // BLK,),
        in_specs=[
            pl.BlockSpec((BLK, D), lambda i: (i, 0)),
            pl.BlockSpec((BLK, 2), lambda i: (i, 0)),
            pl.BlockSpec((1, 1), lambda i: (0, 0)),
        ],
        out_specs=[
            pl.BlockSpec((BLK, D), lambda i: (i, 0)),
            pl.BlockSpec((BLK, 1), lambda i: (i, 0)),
        ],
        out_shape=[
            jax.ShapeDtypeStruct((N, D), jnp.float32),
            jax.ShapeDtypeStruct((N, 1), jnp.float32),
        ],
    )(h, al, c11)


def _stage_g1_body(num_ref, dp_ref, es_ref, b_ref, h1_ref):
    den = dp_ref[:, 0:1] + dp_ref[:, 1:2] + es_ref[...] + 1e-16
    h1_ref[...] = num_ref[...] / den + b_ref[...]


def _stage_g1(num, dpT, es, b_gat):
    return pl.pallas_call(
        _stage_g1_body,
        grid=(N // BLK,),
        in_specs=[
            pl.BlockSpec((BLK, D), lambda i: (i, 0)),
            pl.BlockSpec((BLK, 2), lambda i: (i, 0)),
            pl.BlockSpec((BLK, 1), lambda i: (i, 0)),
            pl.BlockSpec((1, D), lambda i: (0, 0)),
        ],
        out_specs=pl.BlockSpec((BLK, D), lambda i: (i, 0)),
        out_shape=jax.ShapeDtypeStruct((N, D), jnp.float32),
    )(num, dpT, es, b_gat.reshape(1, D))


def _stage_g2_body(h1_ref, agg_ref, cp_ref, wl_ref, bl_ref, wr_ref, wv_ref,
                   out_ref):
    cnt = jnp.maximum(cp_ref[:, 0:1] + cp_ref[:, 1:2], 1.0)
    mean = agg_ref[...] / cnt
    h2 = (jnp.dot(mean, wl_ref[...], preferred_element_type=jnp.float32)
          + bl_ref[...]
          + jnp.dot(h1_ref[...], wr_ref[...], preferred_element_type=jnp.float32))
    out_ref[...] = wv_ref[...][:, 0:1] * h1_ref[...] + wv_ref[...][:, 1:2] * h2


def _stage_g2(h1, agg, cpT, W_l, b_l, W_r, wv):
    return pl.pallas_call(
        _stage_g2_body,
        grid=(N // BLK,),
        in_specs=[
            pl.BlockSpec((BLK, D), lambda i: (i, 0)),
            pl.BlockSpec((BLK, D), lambda i: (i, 0)),
            pl.BlockSpec((BLK, 2), lambda i: (i, 0)),
            pl.BlockSpec((D, D), lambda i: (0, 0)),
            pl.BlockSpec((1, D), lambda i: (0, 0)),
            pl.BlockSpec((D, D), lambda i: (0, 0)),
            pl.BlockSpec((1, 2), lambda i: (0, 0)),
        ],
        out_specs=pl.BlockSpec((BLK, D), lambda i: (i, 0)),
        out_shape=jax.ShapeDtypeStruct((N, D), jnp.float32),
    )(h1, agg, cpT, W_l, b_l.reshape(1, D), W_r, wv)


def _b1_body(als_hbm, ald_hbm, src_hbm, dst_hbm, aspe_hbm, adpe_hbm,
             table_v, idx_v, out_v):
    cid = lax.axis_index("c")
    sid = lax.axis_index("s")
    per_tile = E // NS

    @pl.when(cid == 0)
    def _():
        pltpu.sync_copy(als_hbm, table_v)

    @pl.when(cid == 1)
    def _():
        pltpu.sync_copy(ald_hbm, table_v)

    def chunk(j, _):
        base = sid * per_tile + j * CH1

        @pl.when(cid == 0)
        def _():
            pltpu.sync_copy(src_hbm.at[pl.ds(base, CH1)], idx_v)

        @pl.when(cid == 1)
        def _():
            pltpu.sync_copy(dst_hbm.at[pl.ds(base, CH1)], idx_v)

        @plsc.parallel_loop(0, CH1 // 16, unroll=4)
        def gat(g):
            sl = pl.ds(g * 16, 16)
            out_v[sl] = plsc.load_gather(table_v, [idx_v[sl]])

        @pl.when(cid == 0)
        def _():
            pltpu.sync_copy(out_v, aspe_hbm.at[pl.ds(base, CH1)])

        @pl.when(cid == 1)
        def _():
            pltpu.sync_copy(out_v, adpe_hbm.at[pl.ds(base, CH1)])

        return 0

    lax.fori_loop(0, per_tile // CH1, chunk, 0)


def _stage_b1(als, ald, src, dst):
    f = pl.kernel(
        _b1_body,
        out_type=[
            jax.ShapeDtypeStruct((E,), jnp.float32),
            jax.ShapeDtypeStruct((E,), jnp.float32),
        ],
        mesh=_MESH,
        compiler_params=_SC_PARAMS,
        scratch_types=[
            pltpu.VMEM((N,), jnp.float32),
            pltpu.VMEM((CH1,), jnp.int32),
            pltpu.VMEM((CH1,), jnp.float32),
        ],
    )
    return f(als, ald, src, dst)


def _b2_body(aspe_hbm, adpe_hbm, dst_hbm, c_hbm, z1_hbm,
             ee_hbm, dp_hbm, cp_hbm,
             den_s, cnt_s, asv, adv, dstv, eev, onev, cv, zv):
    cid = lax.axis_index("c")
    sid = lax.axis_index("s")
    per_tile = E // (NC * NS)
    wid = cid * NS + sid

    pltpu.sync_copy(c_hbm, cv)
    zsl = pl.ds(sid * (NPAD // NS), NPAD // NS)
    pltpu.sync_copy(z1_hbm, zv)
    pltpu.sync_copy(zv, den_s.at[zsl])
    pltpu.sync_copy(zv, cnt_s.at[zsl])

    def fill_ones(i, _):
        onev[pl.ds(i * 16, 16)] = jnp.full((16,), 1.0, jnp.float32)
        return 0

    lax.fori_loop(0, CH2 // 16, fill_ones, 0)
    plsc.subcore_barrier()

    cvec = cv[pl.ds(0, 16)]

    def chunk(j, _):
        base = wid * per_tile + j * CH2
        pltpu.sync_copy(aspe_hbm.at[pl.ds(base, CH2)], asv)
        pltpu.sync_copy(adpe_hbm.at[pl.ds(base, CH2)], adv)
        pltpu.sync_copy(dst_hbm.at[pl.ds(base, CH2)], dstv)

        @plsc.parallel_loop(0, CH2 // 16, unroll=4)
        def comp(g):
            sl = pl.ds(g * 16, 16)
            a = asv[sl] + adv[sl]
            a = jnp.where(a >= 0, a, 0.2 * a)
            eev[sl] = jnp.exp(a - cvec)
        pltpu.sync_copy(eev, ee_hbm.at[pl.ds(base, CH2)])
        pltpu.sync_copy(eev, den_s.at[dstv], add=True)
        pltpu.sync_copy(onev, cnt_s.at[dstv], add=True)
        return 0

    lax.fori_loop(0, per_tile // CH2, chunk, 0)
    plsc.subcore_barrier()
    obase = cid * NPAD + sid * (NPAD // NS)
    pltpu.sync_copy(den_s.at[zsl], zv)
    pltpu.sync_copy(zv, dp_hbm.at[pl.ds(obase, NPAD // NS)])
    pltpu.sync_copy(cnt_s.at[zsl], zv)
    pltpu.sync_copy(zv, cp_hbm.at[pl.ds(obase, NPAD // NS)])


def _stage_b2(as_pe, ad_pe, dst, c16, z1d):
    f = pl.kernel(
        _b2_body,
        out_type=[
            jax.ShapeDtypeStruct((E,), jnp.float32),
            jax.ShapeDtypeStruct((NC * NPAD,), jnp.float32),
            jax.ShapeDtypeStruct((NC * NPAD,), jnp.float32),
        ],
        mesh=_MESH,
        compiler_params=_SC_PARAMS,
        scratch_types=[
            pltpu.VMEM_SHARED((NPAD,), jnp.float32),
            pltpu.VMEM_SHARED((NPAD,), jnp.float32),
            pltpu.VMEM((CH2,), jnp.float32),
            pltpu.VMEM((CH2,), jnp.float32),
            pltpu.VMEM((CH2,), jnp.int32),
            pltpu.VMEM((CH2,), jnp.float32),
            pltpu.VMEM((CH2,), jnp.float32),
            pltpu.VMEM((16,), jnp.float32),
            pltpu.VMEM((NPAD // NS,), jnp.float32),
        ],
    )
    return f(as_pe, ad_pe, dst, c16, z1d)


def _d_body(src_hbm, dst_hbm, ee_hbm, h_hbm, als_hbm, ald_hbm, dp_hbm,
            c_hbm, bg_hbm, num_hbm,
            acc_s, idxv, dstv, eev, rows_v, denv, semg, sems):
    cid = lax.axis_index("c")
    sid = lax.axis_index("s")
    per_tile = E // NS
    nch = per_tile // CHD
    ii = _iota16()

    pltpu.sync_copy(c_hbm, denv.at[pl.ds(0, 16)])
    cvec = denv[pl.ds(0, 16)]

    def es_into_denv(nbase):
        pltpu.sync_copy(als_hbm.at[pl.ds(nbase, 400)], eev[0])
        pltpu.sync_copy(ald_hbm.at[pl.ds(nbase, 400)], eev[1])

        @plsc.parallel_loop(0, 25)
        def egrp(g):
            sl = pl.ds(g * 16, 16)
            a = eev[0][sl] + eev[1][sl]
            a = jnp.where(a >= 0, a, 0.2 * a)
            denv[sl] = jnp.exp(a - cvec)

    def init_piece(k, _):
        pp = sid + k * NS

        @pl.when(pp < NH // 400)
        def _():
            nbase = cid * NH + pp * 400
            pltpu.sync_copy(h_hbm.at[pl.ds(nbase, 400)],
                            rows_v[0].at[pl.ds(0, 400)])
            es_into_denv(nbase)

            @plsc.parallel_loop(0, 25)
            def sgrp(g):
                dvec = denv[pl.ds(g * 16, 16)]
                for r in range(16):
                    rr = g * 16 + r
                    d16 = jnp.take(dvec, jnp.full((16,), r, jnp.int32))
                    rows_v[0][rr, pl.ds(0, 16)] = (
                        rows_v[0][rr, pl.ds(0, 16)] * d16)
                    rows_v[0][rr, pl.ds(16, 16)] = (
                        rows_v[0][rr, pl.ds(16, 16)] * d16)

            pltpu.sync_copy(rows_v[0].at[pl.ds(0, 400)],
                            acc_s.at[pl.ds(pp * 400, 400)])
        return 0

    lax.fori_loop(0, pl.cdiv(NH // 400, NS), init_piece, 0)
    plsc.subcore_barrier()

    half_lo = cid * NH

    def load_small(j, b):
        base = sid * per_tile + j * CHD
        pltpu.sync_copy(src_hbm.at[pl.ds(base, CHD)], idxv[b])
        pltpu.sync_copy(dst_hbm.at[pl.ds(base, CHD)], dstv[b])
        pltpu.sync_copy(ee_hbm.at[pl.ds(base, CHD)], eev[b])

    load_small(0, 0)
    pltpu.async_copy(h_hbm.at[idxv[0]], rows_v[0], semg[0])

    def pair(jj, _):
        for b in range(2):
            j = 2 * jj + b
            nb = 1 - b

            @pl.when(j < nch - 1)
            def _():
                @pl.when(j >= 1)
                def _():
                    pltpu.make_async_copy(
                        rows_v[nb], acc_s.at[dstv[nb]], sems[nb]).wait()

                load_small(j + 1, nb)
                pltpu.async_copy(h_hbm.at[idxv[nb]], rows_v[nb], semg[nb])

            pltpu.make_async_copy(h_hbm.at[idxv[b]], rows_v[b], semg[b]).wait()

            @plsc.parallel_loop(0, CHD // 16, unroll=2)
            def grp(g):
                sl = pl.ds(g * 16, 16)
                d = dstv[b][sl] - half_lo
                ok = (d >= 0) & (d < NH)
                dstv[b][sl] = jnp.where(ok, d, NH + ii)
                evec = eev[b][sl]
                for r in range(16):
                    rr = g * 16 + r
                    e16 = jnp.take(evec, jnp.full((16,), r, jnp.int32))
                    rows_v[b][rr, pl.ds(0, 16)] = (
                        rows_v[b][rr, pl.ds(0, 16)] * e16)
                    rows_v[b][rr, pl.ds(16, 16)] = (
                        rows_v[b][rr, pl.ds(16, 16)] * e16)
            pltpu.async_copy(rows_v[b], acc_s.at[dstv[b]], sems[b], add=True)
        return 0

    lax.fori_loop(0, nch // 2, pair, 0)
    pltpu.make_async_copy(rows_v[0], acc_s.at[dstv[0]], sems[0]).wait()
    pltpu.make_async_copy(rows_v[1], acc_s.at[dstv[1]], sems[1]).wait()
    plsc.subcore_barrier()

    pltpu.sync_copy(bg_hbm, denv.at[pl.ds(400, 32)])
    bga = denv[pl.ds(400, 16)]
    bgb = denv[pl.ds(416, 16)]

    def out_piece(k, _):
        pp = sid + k * NS

        @pl.when(pp < NH // 400)
        def _():
            nbase = cid * NH + pp * 400
            pltpu.sync_copy(acc_s.at[pl.ds(pp * 400, 400)],
                            rows_v[0].at[pl.ds(0, 400)])
            es_into_denv(nbase)
            pltpu.sync_copy(dp_hbm.at[pl.ds(nbase, 400)], eev[0])
            pltpu.sync_copy(dp_hbm.at[pl.ds(NPAD + nbase, 400)], eev[1])

            @plsc.parallel_loop(0, 25)
            def dgrp(g):
                sl = pl.ds(g * 16, 16)
                denv[sl] = denv[sl] + eev[0][sl] + eev[1][sl] + 1e-16

            @plsc.parallel_loop(0, 25)
            def rrow(g):
                dvec = denv[pl.ds(g * 16, 16)]
                for r in range(16):
                    rr = g * 16 + r
                    d16 = jnp.take(dvec, jnp.full((16,), r, jnp.int32))
                    rows_v[0][rr, pl.ds(0, 16)] = (
                        rows_v[0][rr, pl.ds(0, 16)] / d16 + bga)
                    rows_v[0][rr, pl.ds(16, 16)] = (
                        rows_v[0][rr, pl.ds(16, 16)] / d16 + bgb)
            pltpu.sync_copy(rows_v[0].at[pl.ds(0, 400)],
                            num_hbm.at[pl.ds(nbase, 400)])
        return 0

    lax.fori_loop(0, pl.cdiv(NH // 400, NS), out_piece, 0)


def _stage_d(src, dst, ee, h, als, ald, dp, c16, b_gat):
    f = pl.kernel(
        _d_body,
        out_type=jax.ShapeDtypeStruct((N, D), jnp.float32),
        mesh=_MESH,
        compiler_params=_SC_PARAMS,
        scratch_types=[
            pltpu.VMEM_SHARED((NH + 16, D), jnp.float32),
            [pltpu.VMEM((CHD,), jnp.int32)] * 2,
            [pltpu.VMEM((CHD,), jnp.int32)] * 2,
            [pltpu.VMEM((CHD,), jnp.float32)] * 2,
            [pltpu.VMEM((CHD, D), jnp.float32)] * 2,
            pltpu.VMEM((432,), jnp.float32),
            [pltpu.SemaphoreType.DMA] * 2,
            [pltpu.SemaphoreType.DMA] * 2,
        ],
    )
    return f(src, dst, ee, h, als, ald, dp, c16, b_gat)


def _f_body(src_hbm, dst_hbm, h1_hbm, z2_hbm, agg_hbm,
            acc_s, idxv, dstv, rows_v, semg, sems):
    cid = lax.axis_index("c")
    sid = lax.axis_index("s")
    per_tile = E // NS
    nch = per_tile // CHD
    ii = _iota16()

    pltpu.sync_copy(z2_hbm, rows_v[0].at[pl.ds(0, 400)])

    def init_piece(k, _):
        pp = sid + k * NS

        @pl.when(pp < NH // 400)
        def _():
            pltpu.sync_copy(rows_v[0].at[pl.ds(0, 400)],
                            acc_s.at[pl.ds(pp * 400, 400)])
        return 0

    lax.fori_loop(0, pl.cdiv(NH // 400, NS), init_piece, 0)
    plsc.subcore_barrier()

    half_lo = cid * NH

    def load_small(j, b):
        base = sid * per_tile + j * CHD
        pltpu.sync_copy(src_hbm.at[pl.ds(base, CHD)], idxv[b])
        pltpu.sync_copy(dst_hbm.at[pl.ds(base, CHD)], dstv[b])

    load_small(0, 0)
    pltpu.async_copy(h1_hbm.at[idxv[0]], rows_v[0], semg[0])

    def pair(jj, _):
        for b in range(2):
            j = 2 * jj + b
            nb = 1 - b

            @pl.when(j < nch - 1)
            def _():
                @pl.when(j >= 1)
                def _():
                    pltpu.make_async_copy(
                        rows_v[nb], acc_s.at[dstv[nb]], sems[nb]).wait()

                load_small(j + 1, nb)
                pltpu.async_copy(h1_hbm.at[idxv[nb]], rows_v[nb], semg[nb])

            pltpu.make_async_copy(h1_hbm.at[idxv[b]], rows_v[b], semg[b]).wait()

            @plsc.parallel_loop(0, CHD // 16, unroll=4)
            def grp(g):
                sl = pl.ds(g * 16, 16)
                d = dstv[b][sl] - half_lo
                ok = (d >= 0) & (d < NH)
                dstv[b][sl] = jnp.where(ok, d, NH + ii)
            pltpu.async_copy(rows_v[b], acc_s.at[dstv[b]], sems[b], add=True)
        return 0

    lax.fori_loop(0, nch // 2, pair, 0)
    pltpu.make_async_copy(rows_v[0], acc_s.at[dstv[0]], sems[0]).wait()
    pltpu.make_async_copy(rows_v[1], acc_s.at[dstv[1]], sems[1]).wait()
    plsc.subcore_barrier()

    def out_piece(k, _):
        pp = sid + k * NS

        @pl.when(pp < NH // 400)
        def _():
            pltpu.sync_copy(acc_s.at[pl.ds(pp * 400, 400)],
                            rows_v[0].at[pl.ds(0, 400)])
            pltpu.sync_copy(rows_v[0].at[pl.ds(0, 400)],
                            agg_hbm.at[pl.ds(cid * NH + pp * 400, 400)])
        return 0

    lax.fori_loop(0, pl.cdiv(NH // 400, NS), out_piece, 0)


def _stage_f(src, dst, h1, z2d):
    f = pl.kernel(
        _f_body,
        out_type=jax.ShapeDtypeStruct((N, D), jnp.float32),
        mesh=_MESH,
        compiler_params=_SC_PARAMS,
        scratch_types=[
            pltpu.VMEM_SHARED((NH + 16, D), jnp.float32),
            [pltpu.VMEM((CHD,), jnp.int32)] * 2,
            [pltpu.VMEM((CHD,), jnp.int32)] * 2,
            [pltpu.VMEM((CHD, D), jnp.float32)] * 2,
            [pltpu.SemaphoreType.DMA] * 2,
            [pltpu.SemaphoreType.DMA] * 2,
        ],
    )
    return f(src, dst, h1, z2d)


def _h_body(src_hbm, dst_hbm, out_hbm, sc_hbm,
            idxv, idxdv, bufa, bufb, scv, tmp, sema, semb):
    cid = lax.axis_index("c")
    sid = lax.axis_index("s")
    per_tile = E // (NC * NS)
    nch = per_tile // CHH
    wid = cid * NS + sid
    ii = _iota16()

    def load_idx(j, b):
        base = wid * per_tile + j * CHH
        pltpu.sync_copy(src_hbm.at[pl.ds(base, CHH)], idxv[b])
        pltpu.sync_copy(dst_hbm.at[pl.ds(base, CHH)], idxdv[b])

    load_idx(0, 0)
    pltpu.async_copy(out_hbm.at[idxv[0]], bufa[0], sema[0])
    pltpu.async_copy(out_hbm.at[idxdv[0]], bufb[0], semb[0])

    def pair(jj, _):
        for b in range(2):
            j = 2 * jj + b
            nb = 1 - b

            @pl.when(j < nch)
            def _():
                @pl.when(j < nch - 1)
                def _():
                    load_idx(j + 1, nb)
                    pltpu.async_copy(out_hbm.at[idxv[nb]], bufa[nb], sema[nb])
                    pltpu.async_copy(out_hbm.at[idxdv[nb]], bufb[nb], semb[nb])

                pltpu.make_async_copy(
                    out_hbm.at[idxv[b]], bufa[b], sema[b]).wait()
                pltpu.make_async_copy(
                    out_hbm.at[idxdv[b]], bufb[b], semb[b]).wait()

                @plsc.parallel_loop(0, CHH // 16, unroll=2)
                def grp(g):
                    for r in range(16):
                        rr = g * 16 + r
                        pa = (bufa[b][rr, pl.ds(0, 16)]
                              * bufb[b][rr, pl.ds(0, 16)]
                              + bufa[b][rr, pl.ds(16, 16)]
                              * bufb[b][rr, pl.ds(16, 16)])
                        tmp[r * 16 + g, pl.ds(0, 16)] = pa
                    accs = [jnp.full((16,), 0.0, jnp.float32)
                            for _ in range(4)]
                    rows16 = ii * 16 + g
                    for c in range(16):
                        v = plsc.load_gather(tmp, [rows16,
                                                   jnp.full((16,), c,
                                                            jnp.int32)])
                        accs[c % 4] = accs[c % 4] + v
                    scv[pl.ds(g * 16, 16)] = (
                        (accs[0] + accs[1]) + (accs[2] + accs[3]))

                base = wid * per_tile + j * CHH
                pltpu.sync_copy(scv, sc_hbm.at[pl.ds(base, CHH)])
        return 0

    lax.fori_loop(0, (nch + 1) // 2, pair, 0)


def _stage_h(src, dst, out):
    f = pl.kernel(
        _h_body,
        out_type=jax.ShapeDtypeStruct((E,), jnp.float32),
        mesh=_MESH,
        compiler_params=_SC_PARAMS,
        scratch_types=[
            [pltpu.VMEM((CHH,), jnp.int32)] * 2,
            [pltpu.VMEM((CHH,), jnp.int32)] * 2,
            [pltpu.VMEM((CHH, D), jnp.float32)] * 2,
            [pltpu.VMEM((CHH, D), jnp.float32)] * 2,
            pltpu.VMEM((CHH,), jnp.float32),
            pltpu.VMEM((CHH, 16), jnp.float32),
            [pltpu.SemaphoreType.DMA] * 2,
            [pltpu.SemaphoreType.DMA] * 2,
        ],
    )
    return f(src, dst, out)


def kernel(x, edge_index, W_gat, att_src, att_dst, b_gat, W_l, b_l, W_r, alpha_buf):
    src, dst = edge_index[0], edge_index[1]

    h, al, am = _stage_a(x, W_gat, att_src, att_dst)
    a = am[0, 0] + am[0, 1]
    c = jnp.where(a >= 0, a, 0.2 * a)

    als = al[:, 0]
    ald = al[:, 1]
    as_pe, ad_pe = _stage_b1(als, ald, src, dst)

    c16 = jnp.full((16,), c, jnp.float32)
    z1d = jnp.zeros((NPAD // NS,), jnp.float32)
    ee, dp, cp = _stage_b2(as_pe, ad_pe, dst, c16, z1d)

    h1 = _stage_d(src, dst, ee, h, als, ald, dp, c16, b_gat)

    z2d = jnp.zeros((400, D), jnp.float32)
    agg = _stage_f(src, dst, h1, z2d)

    cpT = cp.reshape(NC, NPAD)[:, :N].T
    wv3 = jax.nn.softmax(alpha_buf, axis=-1)
    wv = jnp.stack([wv3[0], wv3[2]]).reshape(1, 2)
    out = _stage_g2(h1, agg, cpT, W_l, b_l, W_r, wv)

    return _stage_h(src, dst, out)


# speedup vs baseline: 1.0054x; 1.0054x over previous
import jax
import jax.numpy as jnp
from jax import lax
from jax.experimental import pallas as pl
from jax.experimental.pallas import tpu as pltpu
from jax.experimental.pallas import tpu_sc as plsc

N = 100000
E = 1600000
D = 32
F_IN = 11
BLK = 1000
NC = 2
NS = 16
NPAD = 100096
NH = 50000
RPT = NH // NS

CH1 = 2000
CH2 = 2000
CHD = 400
CHH = 400

_MESH = plsc.VectorSubcoreMesh(core_axis_name="c", subcore_axis_name="s")
_SC_PARAMS = pltpu.CompilerParams(needs_layout_passes=False, use_tc_tiling_on_sc=False)


def _iota16():
    return lax.iota(jnp.int32, 16)


def _stage_a_body(x_ref, w_ref, asrc_ref, adst_ref, h_ref, al_ref, am_ref):
    i = pl.program_id(0)

    @pl.when(i == 0)
    def _():
        am_ref[...] = jnp.full((1, 2), -jnp.inf, jnp.float32)

    h = jnp.dot(x_ref[...], w_ref[...], preferred_element_type=jnp.float32)
    h_ref[...] = h
    a_s = (h * asrc_ref[...]).sum(-1, keepdims=True)
    a_d = (h * adst_ref[...]).sum(-1, keepdims=True)
    al_ref[:, 0:1] = a_s
    al_ref[:, 1:2] = a_d
    am_ref[0:1, 0:1] = jnp.maximum(am_ref[0:1, 0:1], jnp.max(a_s).reshape(1, 1))
    am_ref[0:1, 1:2] = jnp.maximum(am_ref[0:1, 1:2], jnp.max(a_d).reshape(1, 1))


def _stage_a(x, W_gat, att_src, att_dst):
    return pl.pallas_call(
        _stage_a_body,
        grid=(N // BLK,),
        in_specs=[
            pl.BlockSpec((BLK, F_IN), lambda i: (i, 0)),
            pl.BlockSpec((F_IN, D), lambda i: (0, 0)),
            pl.BlockSpec((1, D), lambda i: (0, 0)),
            pl.BlockSpec((1, D), lambda i: (0, 0)),
        ],
        out_specs=[
            pl.BlockSpec((BLK, D), lambda i: (i, 0)),
            pl.BlockSpec((BLK, 2), lambda i: (i, 0)),
            pl.BlockSpec((1, 2), lambda i: (0, 0)),
        ],
        out_shape=[
            jax.ShapeDtypeStruct((N, D), jnp.float32),
            jax.ShapeDtypeStruct((N, 2), jnp.float32),
            jax.ShapeDtypeStruct((1, 2), jnp.float32),
        ],
    )(x, W_gat, att_src.reshape(1, D), att_dst.reshape(1, D))


def _stage_g2_body(h1_ref, agg_ref, cp_ref, wl_ref, bl_ref, wr_ref, wv_ref,
                   out_ref):
    cnt = jnp.maximum(cp_ref[:, 0:1] + cp_ref[:, 1:2], 1.0)
    mean = agg_ref[...] / cnt
    h2 = (jnp.dot(mean, wl_ref[...], preferred_element_type=jnp.float32)
          + bl_ref[...]
          + jnp.dot(h1_ref[...], wr_ref[...], preferred_element_type=jnp.float32))
    out_ref[...] = wv_ref[...][:, 0:1] * h1_ref[...] + wv_ref[...][:, 1:2] * h2


def _stage_g2(h1, agg, cpT, W_l, b_l, W_r, wv):
    return pl.pallas_call(
        _stage_g2_body,
        grid=(N // BLK,),
        in_specs=[
            pl.BlockSpec((BLK, D), lambda i: (i, 0)),
            pl.BlockSpec((BLK, D), lambda i: (i, 0)),
            pl.BlockSpec((BLK, 2), lambda i: (i, 0)),
            pl.BlockSpec((D, D), lambda i: (0, 0)),
            pl.BlockSpec((1, D), lambda i: (0, 0)),
            pl.BlockSpec((D, D), lambda i: (0, 0)),
            pl.BlockSpec((1, 2), lambda i: (0, 0)),
        ],
        out_specs=pl.BlockSpec((BLK, D), lambda i: (i, 0)),
        out_shape=jax.ShapeDtypeStruct((N, D), jnp.float32),
    )(h1, agg, cpT, W_l, b_l.reshape(1, D), W_r, wv)


def _b1_body(als_hbm, ald_hbm, src_hbm, dst_hbm, aspe_hbm, adpe_hbm,
             table_v, idx_v, out_v):
    cid = lax.axis_index("c")
    sid = lax.axis_index("s")
    per_tile = E // NS

    @pl.when(cid == 0)
    def _():
        pltpu.sync_copy(als_hbm, table_v)

    @pl.when(cid == 1)
    def _():
        pltpu.sync_copy(ald_hbm, table_v)

    def chunk(j, _):
        base = sid * per_tile + j * CH1

        @pl.when(cid == 0)
        def _():
            pltpu.sync_copy(src_hbm.at[pl.ds(base, CH1)], idx_v)

        @pl.when(cid == 1)
        def _():
            pltpu.sync_copy(dst_hbm.at[pl.ds(base, CH1)], idx_v)

        @plsc.parallel_loop(0, CH1 // 16, unroll=4)
        def gat(g):
            sl = pl.ds(g * 16, 16)
            out_v[sl] = plsc.load_gather(table_v, [idx_v[sl]])

        @pl.when(cid == 0)
        def _():
            pltpu.sync_copy(out_v, aspe_hbm.at[pl.ds(base, CH1)])

        @pl.when(cid == 1)
        def _():
            pltpu.sync_copy(out_v, adpe_hbm.at[pl.ds(base, CH1)])

        return 0

    lax.fori_loop(0, per_tile // CH1, chunk, 0)


def _stage_b1(als, ald, src, dst):
    f = pl.kernel(
        _b1_body,
        out_type=[
            jax.ShapeDtypeStruct((E,), jnp.float32),
            jax.ShapeDtypeStruct((E,), jnp.float32),
        ],
        mesh=_MESH,
        compiler_params=_SC_PARAMS,
        scratch_types=[
            pltpu.VMEM((N,), jnp.float32),
            pltpu.VMEM((CH1,), jnp.int32),
            pltpu.VMEM((CH1,), jnp.float32),
        ],
    )
    return f(als, ald, src, dst)


def _b2_body(aspe_hbm, adpe_hbm, dst_hbm, c_hbm, z1_hbm,
             ee_hbm, dp_hbm, cp_hbm,
             den_s, cnt_s, asv, adv, dstv, eev, onev, cv, zv):
    cid = lax.axis_index("c")
    sid = lax.axis_index("s")
    per_tile = E // (NC * NS)
    wid = cid * NS + sid

    pltpu.sync_copy(c_hbm, cv)
    zsl = pl.ds(sid * (NPAD // NS), NPAD // NS)
    pltpu.sync_copy(z1_hbm, zv)
    pltpu.sync_copy(zv, den_s.at[zsl])
    pltpu.sync_copy(zv, cnt_s.at[zsl])

    def fill_ones(i, _):
        onev[pl.ds(i * 16, 16)] = jnp.full((16,), 1.0, jnp.float32)
        return 0

    lax.fori_loop(0, CH2 // 16, fill_ones, 0)
    plsc.subcore_barrier()

    cvec = cv[pl.ds(0, 16)]

    def chunk(j, _):
        base = wid * per_tile + j * CH2
        pltpu.sync_copy(aspe_hbm.at[pl.ds(base, CH2)], asv)
        pltpu.sync_copy(adpe_hbm.at[pl.ds(base, CH2)], adv)
        pltpu.sync_copy(dst_hbm.at[pl.ds(base, CH2)], dstv)

        @plsc.parallel_loop(0, CH2 // 16, unroll=4)
        def comp(g):
            sl = pl.ds(g * 16, 16)
            a = asv[sl] + adv[sl]
            a = jnp.where(a >= 0, a, 0.2 * a)
            eev[sl] = jnp.exp(a - cvec)
        pltpu.sync_copy(eev, ee_hbm.at[pl.ds(base, CH2)])
        pltpu.sync_copy(eev, den_s.at[dstv], add=True)
        pltpu.sync_copy(onev, cnt_s.at[dstv], add=True)
        return 0

    lax.fori_loop(0, per_tile // CH2, chunk, 0)
    plsc.subcore_barrier()
    obase = cid * NPAD + sid * (NPAD // NS)
    pltpu.sync_copy(den_s.at[zsl], zv)
    pltpu.sync_copy(zv, dp_hbm.at[pl.ds(obase, NPAD // NS)])
    pltpu.sync_copy(cnt_s.at[zsl], zv)
    pltpu.sync_copy(zv, cp_hbm.at[pl.ds(obase, NPAD // NS)])


def _stage_b2(as_pe, ad_pe, dst, c16, z1d):
    f = pl.kernel(
        _b2_body,
        out_type=[
            jax.ShapeDtypeStruct((E,), jnp.float32),
            jax.ShapeDtypeStruct((NC * NPAD,), jnp.float32),
            jax.ShapeDtypeStruct((NC * NPAD,), jnp.float32),
        ],
        mesh=_MESH,
        compiler_params=_SC_PARAMS,
        scratch_types=[
            pltpu.VMEM_SHARED((NPAD,), jnp.float32),
            pltpu.VMEM_SHARED((NPAD,), jnp.float32),
            pltpu.VMEM((CH2,), jnp.float32),
            pltpu.VMEM((CH2,), jnp.float32),
            pltpu.VMEM((CH2,), jnp.int32),
            pltpu.VMEM((CH2,), jnp.float32),
            pltpu.VMEM((CH2,), jnp.float32),
            pltpu.VMEM((16,), jnp.float32),
            pltpu.VMEM((NPAD // NS,), jnp.float32),
        ],
    )
    return f(as_pe, ad_pe, dst, c16, z1d)


def _d_body(src_hbm, dst_hbm, ee_hbm, h_hbm, als_hbm, ald_hbm, dp_hbm,
            c_hbm, bg_hbm, num_hbm,
            acc_s, idxv, dstv, eev, rows_v, denv, semg, sems):
    cid = lax.axis_index("c")
    sid = lax.axis_index("s")
    per_tile = E // NS
    nch = per_tile // CHD
    ii = _iota16()

    pltpu.sync_copy(c_hbm, denv.at[pl.ds(0, 16)])
    cvec = denv[pl.ds(0, 16)]

    def es_into_denv(nbase):
        pltpu.sync_copy(als_hbm.at[pl.ds(nbase, 400)], eev[0])
        pltpu.sync_copy(ald_hbm.at[pl.ds(nbase, 400)], eev[1])

        @plsc.parallel_loop(0, 25)
        def egrp(g):
            sl = pl.ds(g * 16, 16)
            a = eev[0][sl] + eev[1][sl]
            a = jnp.where(a >= 0, a, 0.2 * a)
            denv[sl] = jnp.exp(a - cvec)

    def init_piece(k, _):
        pp = sid + k * NS

        @pl.when(pp < NH // 400)
        def _():
            nbase = cid * NH + pp * 400
            pltpu.sync_copy(h_hbm.at[pl.ds(nbase, 400)],
                            rows_v[0].at[pl.ds(0, 400)])
            es_into_denv(nbase)

            @plsc.parallel_loop(0, 25)
            def sgrp(g):
                dvec = denv[pl.ds(g * 16, 16)]
                for r in range(16):
                    rr = g * 16 + r
                    d16 = jnp.take(dvec, jnp.full((16,), r, jnp.int32))
                    rows_v[0][rr, pl.ds(0, 16)] = (
                        rows_v[0][rr, pl.ds(0, 16)] * d16)
                    rows_v[0][rr, pl.ds(16, 16)] = (
                        rows_v[0][rr, pl.ds(16, 16)] * d16)

            pltpu.sync_copy(rows_v[0].at[pl.ds(0, 400)],
                            acc_s.at[pl.ds(pp * 400, 400)])
        return 0

    lax.fori_loop(0, pl.cdiv(NH // 400, NS), init_piece, 0)
    plsc.subcore_barrier()

    half_lo = cid * NH

    def load_small(j, b):
        base = sid * per_tile + j * CHD
        pltpu.sync_copy(src_hbm.at[pl.ds(base, CHD)], idxv[b])
        pltpu.sync_copy(dst_hbm.at[pl.ds(base, CHD)], dstv[b])
        pltpu.sync_copy(ee_hbm.at[pl.ds(base, CHD)], eev[b])

    load_small(0, 0)
    pltpu.async_copy(h_hbm.at[idxv[0]], rows_v[0], semg[0])

    def pair(jj, _):
        for b in range(2):
            j = 2 * jj + b
            nb = 1 - b

            @pl.when(j < nch - 1)
            def _():
                @pl.when(j >= 1)
                def _():
                    pltpu.make_async_copy(
                        rows_v[nb], acc_s.at[dstv[nb]], sems[nb]).wait()

                load_small(j + 1, nb)
                pltpu.async_copy(h_hbm.at[idxv[nb]], rows_v[nb], semg[nb])

            pltpu.make_async_copy(h_hbm.at[idxv[b]], rows_v[b], semg[b]).wait()

            @plsc.parallel_loop(0, CHD // 16, unroll=2)
            def grp(g):
                sl = pl.ds(g * 16, 16)
                d = dstv[b][sl] - half_lo
                ok = (d >= 0) & (d < NH)
                dstv[b][sl] = jnp.where(ok, d, NH + ii)
                evec = eev[b][sl]
                for r in range(16):
                    rr = g * 16 + r
                    e16 = jnp.take(evec, jnp.full((16,), r, jnp.int32))
                    rows_v[b][rr, pl.ds(0, 16)] = (
                        rows_v[b][rr, pl.ds(0, 16)] * e16)
                    rows_v[b][rr, pl.ds(16, 16)] = (
                        rows_v[b][rr, pl.ds(16, 16)] * e16)
            pltpu.async_copy(rows_v[b], acc_s.at[dstv[b]], sems[b], add=True)
        return 0

    lax.fori_loop(0, nch // 2, pair, 0)
    pltpu.make_async_copy(rows_v[0], acc_s.at[dstv[0]], sems[0]).wait()
    pltpu.make_async_copy(rows_v[1], acc_s.at[dstv[1]], sems[1]).wait()
    plsc.subcore_barrier()

    pltpu.sync_copy(bg_hbm, denv.at[pl.ds(400, 32)])
    bga = denv[pl.ds(400, 16)]
    bgb = denv[pl.ds(416, 16)]

    def out_piece(k, _):
        pp = sid + k * NS

        @pl.when(pp < NH // 400)
        def _():
            nbase = cid * NH + pp * 400
            pltpu.sync_copy(acc_s.at[pl.ds(pp * 400, 400)],
                            rows_v[0].at[pl.ds(0, 400)])
            es_into_denv(nbase)
            pltpu.sync_copy(dp_hbm.at[pl.ds(nbase, 400)], eev[0])
            pltpu.sync_copy(dp_hbm.at[pl.ds(NPAD + nbase, 400)], eev[1])

            @plsc.parallel_loop(0, 25)
            def dgrp(g):
                sl = pl.ds(g * 16, 16)
                denv[sl] = denv[sl] + eev[0][sl] + eev[1][sl] + 1e-16

            @plsc.parallel_loop(0, 25)
            def rrow(g):
                dvec = denv[pl.ds(g * 16, 16)]
                for r in range(16):
                    rr = g * 16 + r
                    d16 = jnp.take(dvec, jnp.full((16,), r, jnp.int32))
                    rows_v[0][rr, pl.ds(0, 16)] = (
                        rows_v[0][rr, pl.ds(0, 16)] / d16 + bga)
                    rows_v[0][rr, pl.ds(16, 16)] = (
                        rows_v[0][rr, pl.ds(16, 16)] / d16 + bgb)
            pltpu.sync_copy(rows_v[0].at[pl.ds(0, 400)],
                            num_hbm.at[pl.ds(nbase, 400)])
        return 0

    lax.fori_loop(0, pl.cdiv(NH // 400, NS), out_piece, 0)


def _stage_d(src, dst, ee, h, als, ald, dp, c16, b_gat):
    f = pl.kernel(
        _d_body,
        out_type=jax.ShapeDtypeStruct((N, D), jnp.float32),
        mesh=_MESH,
        compiler_params=_SC_PARAMS,
        scratch_types=[
            pltpu.VMEM_SHARED((NH + 16, D), jnp.float32),
            [pltpu.VMEM((CHD,), jnp.int32)] * 2,
            [pltpu.VMEM((CHD,), jnp.int32)] * 2,
            [pltpu.VMEM((CHD,), jnp.float32)] * 2,
            [pltpu.VMEM((CHD, D), jnp.float32)] * 2,
            pltpu.VMEM((432,), jnp.float32),
            [pltpu.SemaphoreType.DMA] * 2,
            [pltpu.SemaphoreType.DMA] * 2,
        ],
    )
    return f(src, dst, ee, h, als, ald, dp, c16, b_gat)


def _f_body(src_hbm, dst_hbm, h1_hbm, z2_hbm, agg_hbm,
            acc_s, idxv, dstv, rows_v, semg, sems):
    cid = lax.axis_index("c")
    sid = lax.axis_index("s")
    per_tile = E // NS
    nch = per_tile // CHD
    ii = _iota16()

    pltpu.sync_copy(z2_hbm, rows_v[0].at[pl.ds(0, 400)])

    def init_piece(k, _):
        pp = sid + k * NS

        @pl.when(pp < NH // 400)
        def _():
            pltpu.sync_copy(rows_v[0].at[pl.ds(0, 400)],
                            acc_s.at[pl.ds(pp * 400, 400)])
        return 0

    lax.fori_loop(0, pl.cdiv(NH // 400, NS), init_piece, 0)
    plsc.subcore_barrier()

    half_lo = cid * NH

    def load_small(j, b):
        base = sid * per_tile + j * CHD
        pltpu.sync_copy(src_hbm.at[pl.ds(base, CHD)], idxv[b])
        pltpu.sync_copy(dst_hbm.at[pl.ds(base, CHD)], dstv[b])

    load_small(0, 0)
    pltpu.async_copy(h1_hbm.at[idxv[0]], rows_v[0], semg[0])

    def pair(jj, _):
        for b in range(2):
            j = 2 * jj + b
            nb = 1 - b

            @pl.when(j < nch - 1)
            def _():
                @pl.when(j >= 1)
                def _():
                    pltpu.make_async_copy(
                        rows_v[nb], acc_s.at[dstv[nb]], sems[nb]).wait()

                load_small(j + 1, nb)
                pltpu.async_copy(h1_hbm.at[idxv[nb]], rows_v[nb], semg[nb])

            pltpu.make_async_copy(h1_hbm.at[idxv[b]], rows_v[b], semg[b]).wait()

            @plsc.parallel_loop(0, CHD // 16, unroll=4)
            def grp(g):
                sl = pl.ds(g * 16, 16)
                d = dstv[b][sl] - half_lo
                ok = (d >= 0) & (d < NH)
                dstv[b][sl] = jnp.where(ok, d, NH + ii)
            pltpu.async_copy(rows_v[b], acc_s.at[dstv[b]], sems[b], add=True)
        return 0

    lax.fori_loop(0, nch // 2, pair, 0)
    pltpu.make_async_copy(rows_v[0], acc_s.at[dstv[0]], sems[0]).wait()
    pltpu.make_async_copy(rows_v[1], acc_s.at[dstv[1]], sems[1]).wait()
    plsc.subcore_barrier()

    def out_piece(k, _):
        pp = sid + k * NS

        @pl.when(pp < NH // 400)
        def _():
            pltpu.sync_copy(acc_s.at[pl.ds(pp * 400, 400)],
                            rows_v[0].at[pl.ds(0, 400)])
            pltpu.sync_copy(rows_v[0].at[pl.ds(0, 400)],
                            agg_hbm.at[pl.ds(cid * NH + pp * 400, 400)])
        return 0

    lax.fori_loop(0, pl.cdiv(NH // 400, NS), out_piece, 0)


def _stage_f(src, dst, h1, z2d):
    f = pl.kernel(
        _f_body,
        out_type=jax.ShapeDtypeStruct((N, D), jnp.float32),
        mesh=_MESH,
        compiler_params=_SC_PARAMS,
        scratch_types=[
            pltpu.VMEM_SHARED((NH + 16, D), jnp.float32),
            [pltpu.VMEM((CHD,), jnp.int32)] * 2,
            [pltpu.VMEM((CHD,), jnp.int32)] * 2,
            [pltpu.VMEM((CHD, D), jnp.float32)] * 2,
            [pltpu.SemaphoreType.DMA] * 2,
            [pltpu.SemaphoreType.DMA] * 2,
        ],
    )
    return f(src, dst, h1, z2d)


def _h_body(src_hbm, dst_hbm, out_hbm, sc_hbm,
            idxv, idxdv, bufa, bufb, scv, tmp, sema, semb):
    cid = lax.axis_index("c")
    sid = lax.axis_index("s")
    per_tile = E // (NC * NS)
    nch = per_tile // CHH
    wid = cid * NS + sid
    ii = _iota16()

    def load_idx(j, b):
        base = wid * per_tile + j * CHH
        pltpu.sync_copy(src_hbm.at[pl.ds(base, CHH)], idxv[b])
        pltpu.sync_copy(dst_hbm.at[pl.ds(base, CHH)], idxdv[b])

    load_idx(0, 0)
    pltpu.async_copy(out_hbm.at[idxv[0]], bufa[0], sema[0])
    pltpu.async_copy(out_hbm.at[idxdv[0]], bufb[0], semb[0])

    def pair(jj, _):
        for b in range(2):
            j = 2 * jj + b
            nb = 1 - b

            @pl.when(j < nch)
            def _():
                @pl.when(j < nch - 1)
                def _():
                    load_idx(j + 1, nb)
                    pltpu.async_copy(out_hbm.at[idxv[nb]], bufa[nb], sema[nb])
                    pltpu.async_copy(out_hbm.at[idxdv[nb]], bufb[nb], semb[nb])

                pltpu.make_async_copy(
                    out_hbm.at[idxv[b]], bufa[b], sema[b]).wait()
                pltpu.make_async_copy(
                    out_hbm.at[idxdv[b]], bufb[b], semb[b]).wait()

                @plsc.parallel_loop(0, CHH // 16)
                def grp(g):
                    for r in range(16):
                        rr = g * 16 + r
                        pa = (bufa[b][rr, pl.ds(0, 16)]
                              * bufb[b][rr, pl.ds(0, 16)]
                              + bufa[b][rr, pl.ds(16, 16)]
                              * bufb[b][rr, pl.ds(16, 16)])
                        tmp[r * 16 + g, pl.ds(0, 16)] = pa
                    accs = [jnp.full((16,), 0.0, jnp.float32)
                            for _ in range(4)]
                    rows16 = ii * 16 + g
                    for c in range(16):
                        v = plsc.load_gather(tmp, [rows16,
                                                   jnp.full((16,), c,
                                                            jnp.int32)])
                        accs[c % 4] = accs[c % 4] + v
                    scv[pl.ds(g * 16, 16)] = (
                        (accs[0] + accs[1]) + (accs[2] + accs[3]))

                base = wid * per_tile + j * CHH
                pltpu.sync_copy(scv, sc_hbm.at[pl.ds(base, CHH)])
        return 0

    lax.fori_loop(0, (nch + 1) // 2, pair, 0)


def _stage_h(src, dst, out):
    f = pl.kernel(
        _h_body,
        out_type=jax.ShapeDtypeStruct((E,), jnp.float32),
        mesh=_MESH,
        compiler_params=_SC_PARAMS,
        scratch_types=[
            [pltpu.VMEM((CHH,), jnp.int32)] * 2,
            [pltpu.VMEM((CHH,), jnp.int32)] * 2,
            [pltpu.VMEM((CHH, D), jnp.float32)] * 2,
            [pltpu.VMEM((CHH, D), jnp.float32)] * 2,
            pltpu.VMEM((CHH,), jnp.float32),
            pltpu.VMEM((CHH, 16), jnp.float32),
            [pltpu.SemaphoreType.DMA] * 2,
            [pltpu.SemaphoreType.DMA] * 2,
        ],
    )
    return f(src, dst, out)


def kernel(x, edge_index, W_gat, att_src, att_dst, b_gat, W_l, b_l, W_r, alpha_buf):
    src, dst = edge_index[0], edge_index[1]

    h, al, am = _stage_a(x, W_gat, att_src, att_dst)
    a = am[0, 0] + am[0, 1]
    c = jnp.where(a >= 0, a, 0.2 * a)

    als = al[:, 0]
    ald = al[:, 1]
    as_pe, ad_pe = _stage_b1(als, ald, src, dst)

    c16 = jnp.full((16,), c, jnp.float32)
    z1d = jnp.zeros((NPAD // NS,), jnp.float32)
    ee, dp, cp = _stage_b2(as_pe, ad_pe, dst, c16, z1d)

    h1 = _stage_d(src, dst, ee, h, als, ald, dp, c16, b_gat)

    z2d = jnp.zeros((400, D), jnp.float32)
    agg = _stage_f(src, dst, h1, z2d)

    cpT = cp.reshape(NC, NPAD)[:, :N].T
    wv3 = jax.nn.softmax(alpha_buf, axis=-1)
    wv = jnp.stack([wv3[0], wv3[2]]).reshape(1, 2)
    out = _stage_g2(h1, agg, cpT, W_l, b_l, W_r, wv)

    return _stage_h(src, dst, out)
